# baseline (speedup 1.0000x reference)
import functools

import jax
import jax.numpy as jnp
from jax import lax
from jax.experimental import pallas as pl
from jax.experimental.pallas import tpu as pltpu
from jax.experimental.pallas import tpu_sc as plsc

NTOKEN = 100000
EMB_DIM = 128
BATCH = 4096
HIST = 50

NC = 2
NS = 16
NW = NC * NS

BLOCK = BATCH // NW
NCHUNK = HIST
NBUF = 7
GAHEAD = 3


def _make_gather():
  mesh = plsc.VectorSubcoreMesh(core_axis_name="c", subcore_axis_name="s")

  @functools.partial(
      pl.kernel,
      mesh=mesh,
      out_type=jax.ShapeDtypeStruct((HIST, BATCH, EMB_DIM), jnp.float32),
      scratch_types=[
          pltpu.VMEM((NCHUNK, BLOCK), jnp.int32),
          pltpu.VMEM((NBUF, BLOCK, EMB_DIM), jnp.float32),
          pltpu.SemaphoreType.DMA,
          pltpu.SemaphoreType.DMA,
      ],
      compiler_params=pltpu.CompilerParams(use_tc_tiling_on_sc=True),
  )
  def gather_kernel(table_hbm, idx_hbm, out_hbm, idx_v, rows_v, gsem, ssem):
    wid = lax.axis_index("s") * NC + lax.axis_index("c")
    col = wid * BLOCK
    pltpu.sync_copy(idx_hbm.at[pl.ds(0, 8), pl.ds(col, BLOCK)],
                    idx_v.at[pl.ds(0, 8)])
    for j in range(GAHEAD):
      pltpu.async_copy(table_hbm.at[idx_v.at[j]], rows_v.at[j], gsem)
    pltpu.sync_copy(idx_hbm.at[pl.ds(8, NCHUNK - 8), pl.ds(col, BLOCK)],
                    idx_v.at[pl.ds(8, NCHUNK - 8)])

    def fill(j):
      pltpu.async_copy(table_hbm.at[idx_v.at[j]],
                       rows_v.at[lax.rem(j, NBUF)], gsem)

    def drain_gather(j):
      pltpu.make_async_copy(table_hbm.at[idx_v.at[j]],
                            rows_v.at[lax.rem(j, NBUF)], gsem).wait()

    def start_store(j):
      pltpu.make_async_copy(rows_v.at[lax.rem(j, NBUF)],
                            out_hbm.at[j, pl.ds(col, BLOCK)], ssem).start()

    def drain_store(j):
      pltpu.make_async_copy(rows_v.at[lax.rem(j, NBUF)],
                            out_hbm.at[j, pl.ds(col, BLOCK)], ssem).wait()

    for j in range(NBUF - GAHEAD):
      fill(j + GAHEAD)
      drain_gather(j)
      start_store(j)

    def step(j, carry):
      drain_store(j - (NBUF - GAHEAD))
      fill(j + GAHEAD)
      drain_gather(j)
      start_store(j)
      return carry

    lax.fori_loop(NBUF - GAHEAD, NCHUNK - GAHEAD, step, 0)

    for j in range(NCHUNK - GAHEAD, NCHUNK):
      drain_store(j - (NBUF - GAHEAD))
      drain_gather(j)
      start_store(j)
    for j in range(NCHUNK - (NBUF - GAHEAD), NCHUNK):
      drain_store(j)

  return gather_kernel


_gather = _make_gather()


def kernel(x, table):
  out = _gather(table, x.T.astype(jnp.int32))
  return out.transpose(1, 0, 2)
